# Initial kernel scaffold; baseline (speedup 1.0000x reference)
import jax
import jax.numpy as jnp
from jax.experimental import pallas as pl


def kernel(x_input, edge_index, edge_weight, W1, b1, W2, b2, W_adj, b_adj, mu, Wd1, bd1, Wd2, bd2, Wd3, bd3, Wm, bm, Wdsp, bdsp, Wpi, bpi):
    raise NotImplementedError("write your pallas kernel here")


# speedup vs baseline: 1.0158x; 1.0158x over previous
import functools

import jax
import jax.numpy as jnp
from jax import lax
from jax.experimental import pallas as pl
from jax.experimental.pallas import tpu as pltpu
from jax.experimental.pallas import tpu_sc as plsc

N = 4096
E = 65536
IN_DIM = 2000
HID = 128
LAT = 15
K = 3
NCLUST = 10

NTILES = 16
CHUNK = 128
NCHUNK = 32
RPT = N // NTILES

_f32 = jnp.float32
_i32 = jnp.int32


def _fill(ref, n16, value):
    def body(i, _):
        ref[pl.ds(i * 16, 16)] = jnp.full((16,), value, _f32)
        return 0
    lax.fori_loop(0, n16, body, 0)


def _scale_chunk(rows, w_t, ci, nblk):
    def body(e, _):
        ws = plsc.load_gather(
            w_t, [jnp.full((16,), ci, _i32), jnp.full((16,), e, _i32)])
        for j in range(nblk):
            sl = pl.ds(j * 16, 16)
            rows[e, sl] = rows[e, sl] * ws
        return 0
    lax.fori_loop(0, CHUNK, body, 0)


def _hops(scale_nblk, p_slices, h0, bufs, src_t, dst_t, w_t, rows, acc, sem,
          r0):
    chain = ((h0, 0, bufs[0]), (bufs[0], 1, bufs[1]), (bufs[1], 2, bufs[2]))
    for h_src, step, dbuf in chain:
        pltpu.sync_copy(p_slices[step](), acc.at[pl.ds(r0, RPT)])
        plsc.subcore_barrier()
        for ci in range(NCHUNK):
            pltpu.async_copy(h_src.at[src_t.at[ci]], rows, sem).wait()
            _scale_chunk(rows, w_t, ci, scale_nblk)
            pltpu.sync_copy(rows, acc.at[dst_t.at[ci]], add=True)
        plsc.subcore_barrier()
        pltpu.sync_copy(acc.at[pl.ds(r0, RPT)], dbuf.at[pl.ds(r0, RPT)])
        plsc.subcore_barrier()


def _sc_wide_body(srcr, dstr, ewr, x3,
                  f1, f2, f3, w_out,
                  src_t, dst_t, w_t, idx_t, rows, zrows, norm_v, stage,
                  ones_v, acc, deg_sp, norm_sp, sem):
    core = lax.axis_index("c")
    sub = lax.axis_index("s")

    @pl.when(core == 0)
    def _():
        t = sub
        r0 = t * RPT
        pltpu.sync_copy(srcr.at[t], src_t)
        pltpu.sync_copy(dstr.at[t], dst_t)
        pltpu.sync_copy(ewr.at[t], w_t)

        _fill(stage, RPT // 16, 0.0)
        _fill(ones_v, CHUNK // 16, 1.0)
        pltpu.sync_copy(stage, deg_sp.at[pl.ds(r0, RPT)])
        plsc.subcore_barrier()
        for ci in range(NCHUNK):
            pltpu.sync_copy(ones_v, deg_sp.at[dst_t.at[ci]], add=True)
        plsc.subcore_barrier()

        pltpu.sync_copy(deg_sp.at[pl.ds(r0, RPT)], stage)

        def nrm(i, _):
            sl = pl.ds(i * 16, 16)
            d = jnp.maximum(stage[sl], 1.0)
            bits = plsc.bitcast(d, _i32)
            y = plsc.bitcast(jnp.int32(0x5F3759DF) - (bits >> 1), _f32)
            hd = 0.5 * d
            for _ in range(3):
                y = y * (1.5 - hd * y * y)
            stage[sl] = y
            return 0
        lax.fori_loop(0, RPT // 16, nrm, 0)
        pltpu.sync_copy(stage, norm_sp.at[pl.ds(r0, RPT)])
        plsc.subcore_barrier()
        pltpu.sync_copy(norm_sp, norm_v)

        def wg(g, _):
            ci = g // 8
            off = (g % 8) * 16
            sl = pl.ds(off, 16)
            sidx = src_t[ci, sl]
            didx = dst_t[ci, sl]
            wv = (w_t[ci, sl]
                  * plsc.load_gather(norm_v, [sidx])
                  * plsc.load_gather(norm_v, [didx]))
            w_t[ci, sl] = wv
            return 0
        lax.fori_loop(0, NCHUNK * 8, wg, 0)
        pltpu.sync_copy(w_t, w_out.at[t])

        def zb(i, _):
            zrows[i // 8, pl.ds((i % 8) * 16, 16)] = jnp.zeros((16,), _f32)
            return 0
        lax.fori_loop(0, CHUNK * 8, zb, 0)

        for fb in range(16):
            base = fb * N

            def aidx(g, _):
                ci = g // 8
                sl = pl.ds((g % 8) * 16, 16)
                idx_t[ci, sl] = src_t[ci, sl] + base
                return 0
            lax.fori_loop(0, NCHUNK * 8, aidx, 0)

            for h_src, dbuf in ((x3, f1), (f1, f2), (f2, f3)):
                pltpu.sync_copy(zrows, acc.at[pl.ds(r0, CHUNK)])
                pltpu.sync_copy(zrows, acc.at[pl.ds(r0 + CHUNK, CHUNK)])
                plsc.subcore_barrier()

                def chunk(ci, _):
                    pltpu.async_copy(h_src.at[idx_t.at[ci]], rows, sem).wait()

                    def sc2(e2, _):
                        for u in range(2):
                            e = e2 * 2 + u
                            ws = plsc.load_gather(
                                w_t, [jnp.full((16,), ci, _i32),
                                      jnp.full((16,), e, _i32)])
                            for j in range(HID // 16):
                                sl = pl.ds(j * 16, 16)
                                rows[e, sl] = rows[e, sl] * ws
                        return 0
                    lax.fori_loop(0, CHUNK // 2, sc2, 0)
                    pltpu.sync_copy(rows, acc.at[dst_t.at[ci]], add=True)
                    return 0
                lax.fori_loop(0, NCHUNK, chunk, 0)
                plsc.subcore_barrier()
                pltpu.sync_copy(acc.at[pl.ds(r0, RPT)],
                                dbuf.at[pl.ds(base + r0, RPT)])
                plsc.subcore_barrier()


def _sc_layer2_body(srcr, dstr, wr, zinit, h0,
                    g1, g2, g3,
                    src_t, dst_t, w_t, rows, acc, sem):
    core = lax.axis_index("c")
    sub = lax.axis_index("s")

    @pl.when(core == 0)
    def _():
        t = sub
        r0 = t * RPT
        pltpu.sync_copy(srcr.at[t], src_t)
        pltpu.sync_copy(dstr.at[t], dst_t)
        pltpu.sync_copy(wr.at[t], w_t)
        p_slices = tuple(
            (lambda: zinit.at[pl.ds(r0, RPT)]) for _ in range(3))
        _hops(HID // 16, p_slices, h0, (g1, g2, g3),
              src_t, dst_t, w_t, rows, acc, sem, r0)


_SC_MESH = dict(core_axis_name="c", subcore_axis_name="s")


def _sc_wide(srcr, dstr, ewr, x3):
    kern = pl.kernel(
        _sc_wide_body,
        out_type=(
            jax.ShapeDtypeStruct((16 * N, HID), _f32),
            jax.ShapeDtypeStruct((16 * N, HID), _f32),
            jax.ShapeDtypeStruct((16 * N, HID), _f32),
            jax.ShapeDtypeStruct((NTILES, NCHUNK, CHUNK), _f32),
        ),
        mesh=plsc.VectorSubcoreMesh(**_SC_MESH),
        compiler_params=pltpu.CompilerParams(needs_layout_passes=False),
        scratch_types=[
            pltpu.VMEM((NCHUNK, CHUNK), _i32),
            pltpu.VMEM((NCHUNK, CHUNK), _i32),
            pltpu.VMEM((NCHUNK, CHUNK), _f32),
            pltpu.VMEM((NCHUNK, CHUNK), _i32),
            pltpu.VMEM((CHUNK, HID), _f32),
            pltpu.VMEM((CHUNK, HID), _f32),
            pltpu.VMEM((N,), _f32),
            pltpu.VMEM((RPT,), _f32),
            pltpu.VMEM((CHUNK,), _f32),
            pltpu.VMEM_SHARED((N, HID), _f32),
            pltpu.VMEM_SHARED((N,), _f32),
            pltpu.VMEM_SHARED((N,), _f32),
            pltpu.SemaphoreType.DMA,
        ],
    )
    return kern(srcr, dstr, ewr, x3)


def _tc_enc_body(x_b, f1_b, f2_b, f3_b, w0, w1, w2, w3, b_ref, enc_ref):
    fb = pl.program_id(1)
    feats = (x_b, f1_b, f2_b, f3_b)
    ws = (w0, w1, w2, w3)
    part = sum(
        jnp.dot(f[0].astype(jnp.bfloat16), w[0].astype(jnp.bfloat16),
                preferred_element_type=_f32)
        for f, w in zip(feats, ws))

    @pl.when(fb == 0)
    def _():
        enc_ref[...] = part + b_ref[...]

    @pl.when(fb > 0)
    def _():
        enc_ref[...] += part


def _tc_enc(x3r, f1r, f2r, f3r, wb, b1):
    bm = 512
    fspec = lambda: pl.BlockSpec((1, bm, HID), lambda r, fb: (fb, r, 0))
    wspec = lambda k: pl.BlockSpec((1, HID, HID), lambda r, fb, k=k: (k * 16 + fb, 0, 0))
    return pl.pallas_call(
        _tc_enc_body,
        grid=(N // bm, 16),
        in_specs=[
            fspec(), fspec(), fspec(), fspec(),
            wspec(0), wspec(1), wspec(2), wspec(3),
            pl.BlockSpec((1, HID), lambda r, fb: (0, 0)),
        ],
        out_specs=pl.BlockSpec((bm, HID), lambda r, fb: (r, 0)),
        out_shape=jax.ShapeDtypeStruct((N, HID), _f32),
        compiler_params=_TC_PARAMS,
    )(x3r, f1r, f2r, f3r, wb, wb, wb, wb, b1)


def _sc_layer2(srcr, dstr, wr, zinit, h0):
    kern = pl.kernel(
        _sc_layer2_body,
        out_type=(
            jax.ShapeDtypeStruct((N, HID), _f32),
            jax.ShapeDtypeStruct((N, HID), _f32),
            jax.ShapeDtypeStruct((N, HID), _f32),
        ),
        mesh=plsc.VectorSubcoreMesh(**_SC_MESH),
        compiler_params=pltpu.CompilerParams(needs_layout_passes=False),
        scratch_types=[
            pltpu.VMEM((NCHUNK, CHUNK), _i32),
            pltpu.VMEM((NCHUNK, CHUNK), _i32),
            pltpu.VMEM((NCHUNK, CHUNK), _f32),
            pltpu.VMEM((CHUNK, HID), _f32),
            pltpu.VMEM_SHARED((N, HID), _f32),
            pltpu.SemaphoreType.DMA,
        ],
    )
    return kern(srcr, dstr, wr, zinit, h0)


_TC_PARAMS = pltpu.CompilerParams(vmem_limit_bytes=100 * 1024 * 1024)


def _tc1_body(x_ref, w_ref, b_ref, p_ref, h0_ref):
    p = jnp.dot(x_ref[...], w_ref[...],
                preferred_element_type=_f32, precision=lax.Precision.HIGHEST) + b_ref[...]
    p_ref[...] = p
    h0_ref[...] = p[:, K * HID:]


def _tc1(x, w1r, b1p):
    bm = 512
    return pl.pallas_call(
        _tc1_body,
        grid=(N // bm,),
        in_specs=[
            pl.BlockSpec((bm, IN_DIM), lambda r: (r, 0)),
            pl.BlockSpec((IN_DIM, (K + 1) * HID), lambda r: (0, 0)),
            pl.BlockSpec((1, (K + 1) * HID), lambda r: (0, 0)),
        ],
        out_specs=[
            pl.BlockSpec((bm, (K + 1) * HID), lambda r: (r, 0)),
            pl.BlockSpec((bm, HID), lambda r: (r, 0)),
        ],
        out_shape=[
            jax.ShapeDtypeStruct((N, (K + 1) * HID), _f32),
            jax.ShapeDtypeStruct((N, HID), _f32),
        ],
        compiler_params=_TC_PARAMS,
    )(x, w1r, b1p)


def _tc2_body(e_ref, g1_ref, g2_ref, g3_ref, w_ref, b_ref, zp_ref):
    cat = jnp.concatenate(
        [e_ref[...], g1_ref[...], g2_ref[...], g3_ref[...]], axis=1)
    z = jnp.dot(cat.astype(jnp.bfloat16), w_ref[...].astype(jnp.bfloat16),
                preferred_element_type=_f32) + b_ref[...]
    zp_ref[...] = jnp.concatenate(
        [z, jnp.zeros((z.shape[0], 1), _f32)], axis=1)


def _tc2(enc, g1, g2, g3, w2, b2):
    bm = 512
    blk = lambda: pl.BlockSpec((bm, HID), lambda r: (r, 0))
    return pl.pallas_call(
        _tc2_body,
        grid=(N // bm,),
        in_specs=[
            blk(), blk(), blk(), blk(),
            pl.BlockSpec(((K + 1) * HID, LAT), lambda r: (0, 0)),
            pl.BlockSpec((1, LAT), lambda r: (0, 0)),
        ],
        out_specs=pl.BlockSpec((bm, 16), lambda r: (r, 0)),
        out_shape=jax.ShapeDtypeStruct((N, 16), _f32),
        compiler_params=_TC_PARAMS,
    )(enc, g1, g2, g3, w2, b2)


def _tc3_body(zp_ref, wa_ref, ba_ref, ubar_ref, zbar_ref):
    z15 = zp_ref[:, :LAT].astype(jnp.bfloat16).astype(_f32)
    wa = wa_ref[...].astype(jnp.bfloat16).astype(_f32)
    ba = ba_ref[...]
    gmat = lax.dot_general(wa, wa, (((1,), (1,)), ((), ())),
                           preferred_element_type=_f32, precision=lax.Precision.HIGHEST)
    cvec = lax.dot_general(wa, ba, (((1,), (1,)), ((), ())),
                           preferred_element_type=_f32, precision=lax.Precision.HIGHEST)
    bb = jnp.sum(ba * ba)
    u = jnp.dot(z15, gmat, preferred_element_type=_f32, precision=lax.Precision.HIGHEST)
    s = jnp.dot(z15, cvec, preferred_element_type=_f32, precision=lax.Precision.HIGHEST)
    ones = jnp.ones((N, 1), _f32)
    zer = jnp.zeros((N, LAT), _f32)
    ubar_ref[...] = jnp.concatenate([u, s + bb, ones, zer], axis=1)
    zbar_ref[...] = jnp.concatenate([z15, ones, s, zer], axis=1)


def _tc3(zp, wa, ba):
    return pl.pallas_call(
        _tc3_body,
        out_shape=[
            jax.ShapeDtypeStruct((N, 32), _f32),
            jax.ShapeDtypeStruct((N, 32), _f32),
        ],
        compiler_params=_TC_PARAMS,
    )(zp, wa, ba)


def _tc4_body(u_ref, z_ref, o_ref):
    o_ref[...] = jax.nn.sigmoid(
        lax.dot_general(u_ref[...], z_ref[...], (((1,), (1,)), ((), ())),
                        preferred_element_type=_f32, precision=lax.Precision.HIGHEST))


def _tc4(ubar, zbar):
    bm, bn = 256, 1024
    return pl.pallas_call(
        _tc4_body,
        grid=(N // bm, N // bn),
        in_specs=[
            pl.BlockSpec((bm, 32), lambda i, j: (i, 0)),
            pl.BlockSpec((bn, 32), lambda i, j: (j, 0)),
        ],
        out_specs=pl.BlockSpec((bm, bn), lambda i, j: (i, j)),
        out_shape=jax.ShapeDtypeStruct((N, N), _f32),
        compiler_params=_TC_PARAMS,
    )(ubar, zbar)


def _tc5_body(zp_ref, mu_ref,
              wd1_ref, bd1_ref, wd2_ref, bd2_ref, wd3_ref, bd3_ref,
              wm_ref, bm_ref, wdsp_ref, bdsp_ref, wpi_ref, bpi_ref,
              mean_ref, disp_ref, pi_ref, q_ref):
    z15 = zp_ref[:, :LAT]
    h1 = jax.nn.relu(jnp.dot(z15, wd1_ref[...],
                             preferred_element_type=_f32, precision=lax.Precision.HIGHEST) + bd1_ref[...])
    h2 = jax.nn.relu(jnp.dot(h1, wd2_ref[...],
                             preferred_element_type=_f32, precision=lax.Precision.HIGHEST) + bd2_ref[...])
    h3 = jax.nn.relu(jnp.dot(h2, wd3_ref[...],
                             preferred_element_type=_f32, precision=lax.Precision.HIGHEST) + bd3_ref[...])
    mean_ref[...] = jnp.clip(
        jnp.exp(jnp.dot(h3, wm_ref[...], preferred_element_type=_f32, precision=lax.Precision.HIGHEST)
                + bm_ref[...]), 1e-5, 1e6)
    disp_ref[...] = jnp.clip(
        jax.nn.softplus(jnp.dot(h3, wdsp_ref[...],
                                preferred_element_type=_f32, precision=lax.Precision.HIGHEST) + bdsp_ref[...]),
        1e-4, 1e4)
    pi_ref[...] = jax.nn.sigmoid(
        jnp.dot(h3, wpi_ref[...], preferred_element_type=_f32, precision=lax.Precision.HIGHEST) + bpi_ref[...])
    mu = mu_ref[...]
    cross = lax.dot_general(z15, mu, (((1,), (1,)), ((), ())),
                            preferred_element_type=_f32, precision=lax.Precision.HIGHEST)
    z2 = jnp.sum(z15 * z15, axis=1, keepdims=True)
    m2 = jnp.sum(mu * mu, axis=1)[None, :]
    dist2 = z2 - 2.0 * cross + m2
    q = 1.0 / (1.0 + dist2)
    q_ref[...] = q / jnp.sum(q, axis=1, keepdims=True)


def _tc5(zp, mu, wd1, bd1, wd2, bd2, wd3, bd3, wm, bm_, wdsp, bdsp, wpi, bpi):
    bm = 512
    full = lambda shape: pl.BlockSpec(shape, lambda r: tuple(0 for _ in shape))
    return pl.pallas_call(
        _tc5_body,
        grid=(N // bm,),
        in_specs=[
            pl.BlockSpec((bm, 16), lambda r: (r, 0)),
            full((NCLUST, LAT)),
            full((LAT, 128)), full((1, 128)),
            full((128, 256)), full((1, 256)),
            full((256, 512)), full((1, 512)),
            full((512, IN_DIM)), full((1, IN_DIM)),
            full((512, IN_DIM)), full((1, IN_DIM)),
            full((512, IN_DIM)), full((1, IN_DIM)),
        ],
        out_specs=[
            pl.BlockSpec((bm, IN_DIM), lambda r: (r, 0)),
            pl.BlockSpec((bm, IN_DIM), lambda r: (r, 0)),
            pl.BlockSpec((bm, IN_DIM), lambda r: (r, 0)),
            pl.BlockSpec((bm, NCLUST), lambda r: (r, 0)),
        ],
        out_shape=[
            jax.ShapeDtypeStruct((N, IN_DIM), _f32),
            jax.ShapeDtypeStruct((N, IN_DIM), _f32),
            jax.ShapeDtypeStruct((N, IN_DIM), _f32),
            jax.ShapeDtypeStruct((N, NCLUST), _f32),
        ],
        compiler_params=_TC_PARAMS,
    )(zp, mu, wd1, bd1, wd2, bd2, wd3, bd3, wm, bm_, wdsp, bdsp, wpi, bpi)


def kernel(x_input, edge_index, edge_weight, W1, b1, W2, b2, W_adj, b_adj,
           mu, Wd1, bd1, Wd2, bd2, Wd3, bd3, Wm, bm, Wdsp, bdsp, Wpi, bpi):
    ei = edge_index.astype(_i32)
    srcr = ei[0].reshape(NTILES, NCHUNK, CHUNK)
    dstr = ei[1].reshape(NTILES, NCHUNK, CHUNK)
    ewr = edge_weight.reshape(NTILES, NCHUNK, CHUNK)

    x3 = jnp.pad(x_input, ((0, 0), (0, 16 * HID - IN_DIM)))
    x3 = x3.reshape(N, 16, HID).transpose(1, 0, 2)
    wb = jnp.pad(W1.reshape(K + 1, IN_DIM, HID),
                 ((0, 0), (0, 16 * HID - IN_DIM), (0, 0)))
    wb = wb.reshape((K + 1) * 16, HID, HID)
    zinit = jnp.zeros((N, HID), _f32)

    f1, f2, f3, w = _sc_wide(srcr, dstr, ewr, x3.reshape(16 * N, HID))
    enc = _tc_enc(x3, f1.reshape(16, N, HID), f2.reshape(16, N, HID),
                  f3.reshape(16, N, HID), wb, b1.reshape(1, -1))
    g1, g2, g3 = _sc_layer2(srcr, dstr, w, zinit, enc)
    zp = _tc2(enc, g1, g2, g3, W2, b2.reshape(1, -1))
    z = zp[:, :LAT]
    ubar, zbar = _tc3(zp, W_adj, b_adj.reshape(1, N))
    adj_out = _tc4(ubar, zbar)
    _mean, _disp, _pi, q = _tc5(zp, mu, Wd1, bd1.reshape(1, -1),
                                Wd2, bd2.reshape(1, -1), Wd3,
                                bd3.reshape(1, -1), Wm, bm.reshape(1, -1),
                                Wdsp, bdsp.reshape(1, -1), Wpi,
                                bpi.reshape(1, -1))
    return (adj_out, z, q, _mean, _disp, _pi)


# speedup vs baseline: 1.7499x; 1.7227x over previous
import functools

import jax
import jax.numpy as jnp
from jax import lax
from jax.experimental import pallas as pl
from jax.experimental.pallas import tpu as pltpu
from jax.experimental.pallas import tpu_sc as plsc

N = 4096
E = 65536
IN_DIM = 2000
HID = 128
LAT = 15
K = 3
NCLUST = 10

NTILES = 16
CHUNK = 128
NCHUNK = 32
RPT = N // NTILES

_f32 = jnp.float32
_i32 = jnp.int32


def _fill(ref, n16, value):
    def body(i, _):
        ref[pl.ds(i * 16, 16)] = jnp.full((16,), value, _f32)
        return 0
    lax.fori_loop(0, n16, body, 0)


def _scale_chunk(rows, w_t, ci, nblk):
    def body(e, _):
        ws = plsc.load_gather(
            w_t, [jnp.full((16,), ci, _i32), jnp.full((16,), e, _i32)])
        for j in range(nblk):
            sl = pl.ds(j * 16, 16)
            rows[e, sl] = rows[e, sl] * ws
        return 0
    lax.fori_loop(0, CHUNK, body, 0)


def _hops(scale_nblk, p_slices, h0, bufs, src_t, dst_t, w_t, rows, acc, sem,
          r0):
    chain = ((h0, 0, bufs[0]), (bufs[0], 1, bufs[1]), (bufs[1], 2, bufs[2]))
    for h_src, step, dbuf in chain:
        pltpu.sync_copy(p_slices[step](), acc.at[pl.ds(r0, RPT)])
        plsc.subcore_barrier()
        for ci in range(NCHUNK):
            pltpu.async_copy(h_src.at[src_t.at[ci]], rows, sem).wait()
            _scale_chunk(rows, w_t, ci, scale_nblk)
            pltpu.sync_copy(rows, acc.at[dst_t.at[ci]], add=True)
        plsc.subcore_barrier()
        pltpu.sync_copy(acc.at[pl.ds(r0, RPT)], dbuf.at[pl.ds(r0, RPT)])
        plsc.subcore_barrier()


def _sc_wide_body(srcr, dstr, ewr, x3,
                  f1, f2, f3, w_out,
                  src_t, dst_t, w_t, idx_t, rows, zrows, norm_v, stage,
                  ones_v, acc, deg_sp, norm_sp, sem):
    core = lax.axis_index("c")
    sub = lax.axis_index("s")

    if True:
        t = sub
        r0 = t * RPT
        pltpu.sync_copy(srcr.at[t], src_t)
        pltpu.sync_copy(dstr.at[t], dst_t)
        pltpu.sync_copy(ewr.at[t], w_t)

        _fill(stage, RPT // 16, 0.0)
        _fill(ones_v, CHUNK // 16, 1.0)
        pltpu.sync_copy(stage, deg_sp.at[pl.ds(r0, RPT)])
        plsc.subcore_barrier()
        for ci in range(NCHUNK):
            pltpu.sync_copy(ones_v, deg_sp.at[dst_t.at[ci]], add=True)
        plsc.subcore_barrier()

        pltpu.sync_copy(deg_sp.at[pl.ds(r0, RPT)], stage)

        def nrm(i, _):
            sl = pl.ds(i * 16, 16)
            d = jnp.maximum(stage[sl], 1.0)
            bits = plsc.bitcast(d, _i32)
            y = plsc.bitcast(jnp.int32(0x5F3759DF) - (bits >> 1), _f32)
            hd = 0.5 * d
            for _ in range(3):
                y = y * (1.5 - hd * y * y)
            stage[sl] = y
            return 0
        lax.fori_loop(0, RPT // 16, nrm, 0)
        pltpu.sync_copy(stage, norm_sp.at[pl.ds(r0, RPT)])
        plsc.subcore_barrier()
        pltpu.sync_copy(norm_sp, norm_v)

        def wg(g, _):
            ci = g // 8
            off = (g % 8) * 16
            sl = pl.ds(off, 16)
            sidx = src_t[ci, sl]
            didx = dst_t[ci, sl]
            wv = (w_t[ci, sl]
                  * plsc.load_gather(norm_v, [sidx])
                  * plsc.load_gather(norm_v, [didx]))
            w_t[ci, sl] = wv
            return 0
        lax.fori_loop(0, NCHUNK * 8, wg, 0)

        @pl.when(core == 0)
        def _():
            pltpu.sync_copy(w_t, w_out.at[t])

        def zb(i, _):
            zrows[i // 8, pl.ds((i % 8) * 16, 16)] = jnp.zeros((16,), _f32)
            return 0
        lax.fori_loop(0, CHUNK * 8, zb, 0)

        for fbh in range(8):
            base = (core * 8 + fbh) * N

            def aidx(g, _):
                ci = g // 8
                sl = pl.ds((g % 8) * 16, 16)
                idx_t[ci, sl] = src_t[ci, sl] + base
                return 0
            lax.fori_loop(0, NCHUNK * 8, aidx, 0)

            for h_src, dbuf in ((x3, f1), (f1, f2), (f2, f3)):
                pltpu.sync_copy(zrows, acc.at[pl.ds(r0, CHUNK)])
                pltpu.sync_copy(zrows, acc.at[pl.ds(r0 + CHUNK, CHUNK)])
                plsc.subcore_barrier()

                def chunk(ci, _):
                    pltpu.async_copy(h_src.at[idx_t.at[ci]], rows, sem).wait()

                    def sc2(e2, _):
                        for u in range(2):
                            e = e2 * 2 + u
                            ws = plsc.load_gather(
                                w_t, [jnp.full((16,), ci, _i32),
                                      jnp.full((16,), e, _i32)])
                            for j in range(HID // 16):
                                sl = pl.ds(j * 16, 16)
                                rows[e, sl] = rows[e, sl] * ws
                        return 0
                    lax.fori_loop(0, CHUNK // 2, sc2, 0)
                    pltpu.sync_copy(rows, acc.at[dst_t.at[ci]], add=True)
                    return 0
                lax.fori_loop(0, NCHUNK, chunk, 0)
                plsc.subcore_barrier()
                pltpu.sync_copy(acc.at[pl.ds(r0, RPT)],
                                dbuf.at[pl.ds(base + r0, RPT)])
                plsc.subcore_barrier()


def _sc_layer2_body(srcr, dstr, wr, zinit, h0,
                    g1, g2, g3,
                    src_t, dst_t, w_t, rows, acc, sem):
    core = lax.axis_index("c")
    sub = lax.axis_index("s")

    @pl.when(core == 0)
    def _():
        t = sub
        r0 = t * RPT
        pltpu.sync_copy(srcr.at[t], src_t)
        pltpu.sync_copy(dstr.at[t], dst_t)
        pltpu.sync_copy(wr.at[t], w_t)
        p_slices = tuple(
            (lambda: zinit.at[pl.ds(r0, RPT)]) for _ in range(3))
        _hops(HID // 16, p_slices, h0, (g1, g2, g3),
              src_t, dst_t, w_t, rows, acc, sem, r0)


_SC_MESH = dict(core_axis_name="c", subcore_axis_name="s")


def _sc_wide(srcr, dstr, ewr, x3):
    kern = pl.kernel(
        _sc_wide_body,
        out_type=(
            jax.ShapeDtypeStruct((16 * N, HID), _f32),
            jax.ShapeDtypeStruct((16 * N, HID), _f32),
            jax.ShapeDtypeStruct((16 * N, HID), _f32),
            jax.ShapeDtypeStruct((NTILES, NCHUNK, CHUNK), _f32),
        ),
        mesh=plsc.VectorSubcoreMesh(**_SC_MESH),
        compiler_params=pltpu.CompilerParams(needs_layout_passes=False),
        scratch_types=[
            pltpu.VMEM((NCHUNK, CHUNK), _i32),
            pltpu.VMEM((NCHUNK, CHUNK), _i32),
            pltpu.VMEM((NCHUNK, CHUNK), _f32),
            pltpu.VMEM((NCHUNK, CHUNK), _i32),
            pltpu.VMEM((CHUNK, HID), _f32),
            pltpu.VMEM((CHUNK, HID), _f32),
            pltpu.VMEM((N,), _f32),
            pltpu.VMEM((RPT,), _f32),
            pltpu.VMEM((CHUNK,), _f32),
            pltpu.VMEM_SHARED((N, HID), _f32),
            pltpu.VMEM_SHARED((N,), _f32),
            pltpu.VMEM_SHARED((N,), _f32),
            pltpu.SemaphoreType.DMA,
        ],
    )
    return kern(srcr, dstr, ewr, x3)


def _tc_enc_body(x_b, f1_b, f2_b, f3_b, w0, w1, w2, w3, b_ref, enc_ref):
    fb = pl.program_id(1)
    feats = (x_b, f1_b, f2_b, f3_b)
    ws = (w0, w1, w2, w3)
    part = sum(
        jnp.dot(f[0].astype(jnp.bfloat16), w[0].astype(jnp.bfloat16),
                preferred_element_type=_f32)
        for f, w in zip(feats, ws))

    @pl.when(fb == 0)
    def _():
        enc_ref[...] = part + b_ref[...]

    @pl.when(fb > 0)
    def _():
        enc_ref[...] += part


def _tc_enc(x3r, f1r, f2r, f3r, wb, b1):
    bm = 512
    fspec = lambda: pl.BlockSpec((1, bm, HID), lambda r, fb: (fb, r, 0))
    wspec = lambda k: pl.BlockSpec((1, HID, HID), lambda r, fb, k=k: (k * 16 + fb, 0, 0))
    return pl.pallas_call(
        _tc_enc_body,
        grid=(N // bm, 16),
        in_specs=[
            fspec(), fspec(), fspec(), fspec(),
            wspec(0), wspec(1), wspec(2), wspec(3),
            pl.BlockSpec((1, HID), lambda r, fb: (0, 0)),
        ],
        out_specs=pl.BlockSpec((bm, HID), lambda r, fb: (r, 0)),
        out_shape=jax.ShapeDtypeStruct((N, HID), _f32),
        compiler_params=_TC_PARAMS,
    )(x3r, f1r, f2r, f3r, wb, wb, wb, wb, b1)


def _sc_layer2(srcr, dstr, wr, zinit, h0):
    kern = pl.kernel(
        _sc_layer2_body,
        out_type=(
            jax.ShapeDtypeStruct((N, HID), _f32),
            jax.ShapeDtypeStruct((N, HID), _f32),
            jax.ShapeDtypeStruct((N, HID), _f32),
        ),
        mesh=plsc.VectorSubcoreMesh(**_SC_MESH),
        compiler_params=pltpu.CompilerParams(needs_layout_passes=False),
        scratch_types=[
            pltpu.VMEM((NCHUNK, CHUNK), _i32),
            pltpu.VMEM((NCHUNK, CHUNK), _i32),
            pltpu.VMEM((NCHUNK, CHUNK), _f32),
            pltpu.VMEM((CHUNK, HID), _f32),
            pltpu.VMEM_SHARED((N, HID), _f32),
            pltpu.SemaphoreType.DMA,
        ],
    )
    return kern(srcr, dstr, wr, zinit, h0)


_TC_PARAMS = pltpu.CompilerParams(vmem_limit_bytes=100 * 1024 * 1024)


def _tc1_body(x_ref, w_ref, b_ref, p_ref, h0_ref):
    p = jnp.dot(x_ref[...], w_ref[...],
                preferred_element_type=_f32, precision=lax.Precision.HIGHEST) + b_ref[...]
    p_ref[...] = p
    h0_ref[...] = p[:, K * HID:]


def _tc1(x, w1r, b1p):
    bm = 512
    return pl.pallas_call(
        _tc1_body,
        grid=(N // bm,),
        in_specs=[
            pl.BlockSpec((bm, IN_DIM), lambda r: (r, 0)),
            pl.BlockSpec((IN_DIM, (K + 1) * HID), lambda r: (0, 0)),
            pl.BlockSpec((1, (K + 1) * HID), lambda r: (0, 0)),
        ],
        out_specs=[
            pl.BlockSpec((bm, (K + 1) * HID), lambda r: (r, 0)),
            pl.BlockSpec((bm, HID), lambda r: (r, 0)),
        ],
        out_shape=[
            jax.ShapeDtypeStruct((N, (K + 1) * HID), _f32),
            jax.ShapeDtypeStruct((N, HID), _f32),
        ],
        compiler_params=_TC_PARAMS,
    )(x, w1r, b1p)


def _tc2_body(e_ref, g1_ref, g2_ref, g3_ref, w_ref, b_ref, zp_ref):
    cat = jnp.concatenate(
        [e_ref[...], g1_ref[...], g2_ref[...], g3_ref[...]], axis=1)
    z = jnp.dot(cat.astype(jnp.bfloat16), w_ref[...].astype(jnp.bfloat16),
                preferred_element_type=_f32) + b_ref[...]
    zp_ref[...] = jnp.concatenate(
        [z, jnp.zeros((z.shape[0], 1), _f32)], axis=1)


def _tc2(enc, g1, g2, g3, w2, b2):
    bm = 512
    blk = lambda: pl.BlockSpec((bm, HID), lambda r: (r, 0))
    return pl.pallas_call(
        _tc2_body,
        grid=(N // bm,),
        in_specs=[
            blk(), blk(), blk(), blk(),
            pl.BlockSpec(((K + 1) * HID, LAT), lambda r: (0, 0)),
            pl.BlockSpec((1, LAT), lambda r: (0, 0)),
        ],
        out_specs=pl.BlockSpec((bm, 16), lambda r: (r, 0)),
        out_shape=jax.ShapeDtypeStruct((N, 16), _f32),
        compiler_params=_TC_PARAMS,
    )(enc, g1, g2, g3, w2, b2)


def _tc3_body(zp_ref, wa_ref, ba_ref, ubar_ref, zbar_ref):
    z15 = zp_ref[:, :LAT].astype(jnp.bfloat16).astype(_f32)
    wa = wa_ref[...].astype(jnp.bfloat16).astype(_f32)
    ba = ba_ref[...]
    gmat = lax.dot_general(wa, wa, (((1,), (1,)), ((), ())),
                           preferred_element_type=_f32, precision=lax.Precision.HIGHEST)
    cvec = lax.dot_general(wa, ba, (((1,), (1,)), ((), ())),
                           preferred_element_type=_f32, precision=lax.Precision.HIGHEST)
    bb = jnp.sum(ba * ba)
    u = jnp.dot(z15, gmat, preferred_element_type=_f32, precision=lax.Precision.HIGHEST)
    s = jnp.dot(z15, cvec, preferred_element_type=_f32, precision=lax.Precision.HIGHEST)
    ones = jnp.ones((N, 1), _f32)
    zer = jnp.zeros((N, LAT), _f32)
    ubar_ref[...] = jnp.concatenate([u, s + bb, ones, zer], axis=1)
    zbar_ref[...] = jnp.concatenate([z15, ones, s, zer], axis=1)


def _tc3(zp, wa, ba):
    return pl.pallas_call(
        _tc3_body,
        out_shape=[
            jax.ShapeDtypeStruct((N, 32), _f32),
            jax.ShapeDtypeStruct((N, 32), _f32),
        ],
        compiler_params=_TC_PARAMS,
    )(zp, wa, ba)


def _tc4_body(u_ref, z_ref, o_ref):
    o_ref[...] = jax.nn.sigmoid(
        lax.dot_general(u_ref[...], z_ref[...], (((1,), (1,)), ((), ())),
                        preferred_element_type=_f32, precision=lax.Precision.HIGHEST))


def _tc4(ubar, zbar):
    bm, bn = 256, 1024
    return pl.pallas_call(
        _tc4_body,
        grid=(N // bm, N // bn),
        in_specs=[
            pl.BlockSpec((bm, 32), lambda i, j: (i, 0)),
            pl.BlockSpec((bn, 32), lambda i, j: (j, 0)),
        ],
        out_specs=pl.BlockSpec((bm, bn), lambda i, j: (i, j)),
        out_shape=jax.ShapeDtypeStruct((N, N), _f32),
        compiler_params=_TC_PARAMS,
    )(ubar, zbar)


def _tc5_body(zp_ref, mu_ref,
              wd1_ref, bd1_ref, wd2_ref, bd2_ref, wd3_ref, bd3_ref,
              wm_ref, bm_ref, wdsp_ref, bdsp_ref, wpi_ref, bpi_ref,
              mean_ref, disp_ref, pi_ref, q_ref):
    z15 = zp_ref[:, :LAT]
    h1 = jax.nn.relu(jnp.dot(z15, wd1_ref[...],
                             preferred_element_type=_f32, precision=lax.Precision.HIGHEST) + bd1_ref[...])
    h2 = jax.nn.relu(jnp.dot(h1, wd2_ref[...],
                             preferred_element_type=_f32, precision=lax.Precision.HIGHEST) + bd2_ref[...])
    h3 = jax.nn.relu(jnp.dot(h2, wd3_ref[...],
                             preferred_element_type=_f32, precision=lax.Precision.HIGHEST) + bd3_ref[...])
    mean_ref[...] = jnp.clip(
        jnp.exp(jnp.dot(h3, wm_ref[...], preferred_element_type=_f32, precision=lax.Precision.HIGHEST)
                + bm_ref[...]), 1e-5, 1e6)
    disp_ref[...] = jnp.clip(
        jax.nn.softplus(jnp.dot(h3, wdsp_ref[...],
                                preferred_element_type=_f32, precision=lax.Precision.HIGHEST) + bdsp_ref[...]),
        1e-4, 1e4)
    pi_ref[...] = jax.nn.sigmoid(
        jnp.dot(h3, wpi_ref[...], preferred_element_type=_f32, precision=lax.Precision.HIGHEST) + bpi_ref[...])
    mu = mu_ref[...]
    cross = lax.dot_general(z15, mu, (((1,), (1,)), ((), ())),
                            preferred_element_type=_f32, precision=lax.Precision.HIGHEST)
    z2 = jnp.sum(z15 * z15, axis=1, keepdims=True)
    m2 = jnp.sum(mu * mu, axis=1)[None, :]
    dist2 = z2 - 2.0 * cross + m2
    q = 1.0 / (1.0 + dist2)
    q_ref[...] = q / jnp.sum(q, axis=1, keepdims=True)


def _tc5(zp, mu, wd1, bd1, wd2, bd2, wd3, bd3, wm, bm_, wdsp, bdsp, wpi, bpi):
    bm = 512
    full = lambda shape: pl.BlockSpec(shape, lambda r: tuple(0 for _ in shape))
    return pl.pallas_call(
        _tc5_body,
        grid=(N // bm,),
        in_specs=[
            pl.BlockSpec((bm, 16), lambda r: (r, 0)),
            full((NCLUST, LAT)),
            full((LAT, 128)), full((1, 128)),
            full((128, 256)), full((1, 256)),
            full((256, 512)), full((1, 512)),
            full((512, IN_DIM)), full((1, IN_DIM)),
            full((512, IN_DIM)), full((1, IN_DIM)),
            full((512, IN_DIM)), full((1, IN_DIM)),
        ],
        out_specs=[
            pl.BlockSpec((bm, IN_DIM), lambda r: (r, 0)),
            pl.BlockSpec((bm, IN_DIM), lambda r: (r, 0)),
            pl.BlockSpec((bm, IN_DIM), lambda r: (r, 0)),
            pl.BlockSpec((bm, NCLUST), lambda r: (r, 0)),
        ],
        out_shape=[
            jax.ShapeDtypeStruct((N, IN_DIM), _f32),
            jax.ShapeDtypeStruct((N, IN_DIM), _f32),
            jax.ShapeDtypeStruct((N, IN_DIM), _f32),
            jax.ShapeDtypeStruct((N, NCLUST), _f32),
        ],
        compiler_params=_TC_PARAMS,
    )(zp, mu, wd1, bd1, wd2, bd2, wd3, bd3, wm, bm_, wdsp, bdsp, wpi, bpi)


def kernel(x_input, edge_index, edge_weight, W1, b1, W2, b2, W_adj, b_adj,
           mu, Wd1, bd1, Wd2, bd2, Wd3, bd3, Wm, bm, Wdsp, bdsp, Wpi, bpi):
    ei = edge_index.astype(_i32)
    srcr = ei[0].reshape(NTILES, NCHUNK, CHUNK)
    dstr = ei[1].reshape(NTILES, NCHUNK, CHUNK)
    ewr = edge_weight.reshape(NTILES, NCHUNK, CHUNK)

    x3 = jnp.pad(x_input, ((0, 0), (0, 16 * HID - IN_DIM)))
    x3 = x3.reshape(N, 16, HID).transpose(1, 0, 2)
    wb = jnp.pad(W1.reshape(K + 1, IN_DIM, HID),
                 ((0, 0), (0, 16 * HID - IN_DIM), (0, 0)))
    wb = wb.reshape((K + 1) * 16, HID, HID)
    zinit = jnp.zeros((N, HID), _f32)

    f1, f2, f3, w = _sc_wide(srcr, dstr, ewr, x3.reshape(16 * N, HID))
    enc = _tc_enc(x3, f1.reshape(16, N, HID), f2.reshape(16, N, HID),
                  f3.reshape(16, N, HID), wb, b1.reshape(1, -1))
    g1, g2, g3 = _sc_layer2(srcr, dstr, w, zinit, enc)
    zp = _tc2(enc, g1, g2, g3, W2, b2.reshape(1, -1))
    z = zp[:, :LAT]
    ubar, zbar = _tc3(zp, W_adj, b_adj.reshape(1, N))
    adj_out = _tc4(ubar, zbar)
    _mean, _disp, _pi, q = _tc5(zp, mu, Wd1, bd1.reshape(1, -1),
                                Wd2, bd2.reshape(1, -1), Wd3,
                                bd3.reshape(1, -1), Wm, bm.reshape(1, -1),
                                Wdsp, bdsp.reshape(1, -1), Wpi,
                                bpi.reshape(1, -1))
    return (adj_out, z, q, _mean, _disp, _pi)


# speedup vs baseline: 1.7546x; 1.0027x over previous
import functools

import jax
import jax.numpy as jnp
from jax import lax
from jax.experimental import pallas as pl
from jax.experimental.pallas import tpu as pltpu
from jax.experimental.pallas import tpu_sc as plsc

N = 4096
E = 65536
IN_DIM = 2000
HID = 128
LAT = 15
K = 3
NCLUST = 10

NTILES = 16
CHUNK = 128
NCHUNK = 32
RPT = N // NTILES

_f32 = jnp.float32
_i32 = jnp.int32


def _fill(ref, n16, value):
    def body(i, _):
        ref[pl.ds(i * 16, 16)] = jnp.full((16,), value, _f32)
        return 0
    lax.fori_loop(0, n16, body, 0)


def _scale_chunk(rows, w_t, ci, nblk):
    def body(e, _):
        ws = plsc.load_gather(
            w_t, [jnp.full((16,), ci, _i32), jnp.full((16,), e, _i32)])
        for j in range(nblk):
            sl = pl.ds(j * 16, 16)
            rows[e, sl] = rows[e, sl] * ws
        return 0
    lax.fori_loop(0, CHUNK, body, 0)


def _hops(scale_nblk, p_slices, h0, bufs, src_t, dst_t, w_t, rows, acc, sem,
          r0):
    chain = ((h0, 0, bufs[0]), (bufs[0], 1, bufs[1]), (bufs[1], 2, bufs[2]))
    for h_src, step, dbuf in chain:
        pltpu.sync_copy(p_slices[step](), acc.at[pl.ds(r0, RPT)])
        plsc.subcore_barrier()
        for ci in range(NCHUNK):
            pltpu.async_copy(h_src.at[src_t.at[ci]], rows, sem).wait()
            _scale_chunk(rows, w_t, ci, scale_nblk)
            pltpu.sync_copy(rows, acc.at[dst_t.at[ci]], add=True)
        plsc.subcore_barrier()
        pltpu.sync_copy(acc.at[pl.ds(r0, RPT)], dbuf.at[pl.ds(r0, RPT)])
        plsc.subcore_barrier()


def _sc_wide_body(srcr, dstr, ewr, x3,
                  f1, f2, f3, w_out,
                  src_t, dst_t, w_t, idx_t, rows, zrows, norm_v, stage,
                  ones_v, acc, deg_sp, norm_sp, sem):
    core = lax.axis_index("c")
    sub = lax.axis_index("s")

    if True:
        t = sub
        r0 = t * RPT
        pltpu.sync_copy(srcr.at[t], src_t)
        pltpu.sync_copy(dstr.at[t], dst_t)
        pltpu.sync_copy(ewr.at[t], w_t)

        _fill(stage, RPT // 16, 0.0)
        _fill(ones_v, CHUNK // 16, 1.0)
        pltpu.sync_copy(stage, deg_sp.at[pl.ds(r0, RPT)])
        plsc.subcore_barrier()
        for ci in range(NCHUNK):
            pltpu.sync_copy(ones_v, deg_sp.at[dst_t.at[ci]], add=True)
        plsc.subcore_barrier()

        pltpu.sync_copy(deg_sp.at[pl.ds(r0, RPT)], stage)

        def nrm(i, _):
            sl = pl.ds(i * 16, 16)
            d = jnp.maximum(stage[sl], 1.0)
            bits = plsc.bitcast(d, _i32)
            y = plsc.bitcast(jnp.int32(0x5F3759DF) - (bits >> 1), _f32)
            hd = 0.5 * d
            for _ in range(3):
                y = y * (1.5 - hd * y * y)
            stage[sl] = y
            return 0
        lax.fori_loop(0, RPT // 16, nrm, 0)
        pltpu.sync_copy(stage, norm_sp.at[pl.ds(r0, RPT)])
        plsc.subcore_barrier()
        pltpu.sync_copy(norm_sp, norm_v)

        def wg(g, _):
            ci = g // 8
            off = (g % 8) * 16
            sl = pl.ds(off, 16)
            sidx = src_t[ci, sl]
            didx = dst_t[ci, sl]
            wv = (w_t[ci, sl]
                  * plsc.load_gather(norm_v, [sidx])
                  * plsc.load_gather(norm_v, [didx]))
            w_t[ci, sl] = wv
            return 0
        lax.fori_loop(0, NCHUNK * 8, wg, 0)

        @pl.when(core == 0)
        def _():
            pltpu.sync_copy(w_t, w_out.at[t])

        def zb(i, _):
            zrows[i // 8, pl.ds((i % 8) * 16, 16)] = jnp.zeros((16,), _f32)
            return 0
        lax.fori_loop(0, CHUNK * 8, zb, 0)

        for fbh in range(8):
            base = (core * 8 + fbh) * N

            def aidx(g, _):
                ci = g // 8
                sl = pl.ds((g % 8) * 16, 16)
                idx_t[ci, sl] = src_t[ci, sl] + base
                return 0
            lax.fori_loop(0, NCHUNK * 8, aidx, 0)

            for h_src, dbuf in ((x3, f1), (f1, f2), (f2, f3)):
                pltpu.sync_copy(zrows, acc.at[pl.ds(r0, CHUNK)])
                pltpu.sync_copy(zrows, acc.at[pl.ds(r0 + CHUNK, CHUNK)])
                plsc.subcore_barrier()

                def chunk(ci, _):
                    pltpu.async_copy(h_src.at[idx_t.at[ci]], rows, sem).wait()

                    def sc4(e4, _):
                        for u in range(4):
                            e = e4 * 4 + u
                            ws = plsc.load_gather(
                                w_t, [jnp.full((16,), ci, _i32),
                                      jnp.full((16,), e, _i32)])
                            for j in range(HID // 16):
                                sl = pl.ds(j * 16, 16)
                                rows[e, sl] = rows[e, sl] * ws
                        return 0
                    lax.fori_loop(0, CHUNK // 4, sc4, 0)
                    pltpu.sync_copy(rows, acc.at[dst_t.at[ci]], add=True)
                    return 0
                lax.fori_loop(0, NCHUNK, chunk, 0)
                plsc.subcore_barrier()
                pltpu.sync_copy(acc.at[pl.ds(r0, RPT)],
                                dbuf.at[pl.ds(base + r0, RPT)])
                plsc.subcore_barrier()


def _sc_layer2_body(srcr, dstr, wr, zinit, h0,
                    g1, g2, g3,
                    src_t, dst_t, w_t, rows, acc, sem):
    core = lax.axis_index("c")
    sub = lax.axis_index("s")

    @pl.when(core == 0)
    def _():
        t = sub
        r0 = t * RPT
        pltpu.sync_copy(srcr.at[t], src_t)
        pltpu.sync_copy(dstr.at[t], dst_t)
        pltpu.sync_copy(wr.at[t], w_t)
        p_slices = tuple(
            (lambda: zinit.at[pl.ds(r0, RPT)]) for _ in range(3))
        _hops(HID // 16, p_slices, h0, (g1, g2, g3),
              src_t, dst_t, w_t, rows, acc, sem, r0)


_SC_MESH = dict(core_axis_name="c", subcore_axis_name="s")


def _sc_wide(srcr, dstr, ewr, x3):
    kern = pl.kernel(
        _sc_wide_body,
        out_type=(
            jax.ShapeDtypeStruct((16 * N, HID), _f32),
            jax.ShapeDtypeStruct((16 * N, HID), _f32),
            jax.ShapeDtypeStruct((16 * N, HID), _f32),
            jax.ShapeDtypeStruct((NTILES, NCHUNK, CHUNK), _f32),
        ),
        mesh=plsc.VectorSubcoreMesh(**_SC_MESH),
        compiler_params=pltpu.CompilerParams(needs_layout_passes=False),
        scratch_types=[
            pltpu.VMEM((NCHUNK, CHUNK), _i32),
            pltpu.VMEM((NCHUNK, CHUNK), _i32),
            pltpu.VMEM((NCHUNK, CHUNK), _f32),
            pltpu.VMEM((NCHUNK, CHUNK), _i32),
            pltpu.VMEM((CHUNK, HID), _f32),
            pltpu.VMEM((CHUNK, HID), _f32),
            pltpu.VMEM((N,), _f32),
            pltpu.VMEM((RPT,), _f32),
            pltpu.VMEM((CHUNK,), _f32),
            pltpu.VMEM_SHARED((N, HID), _f32),
            pltpu.VMEM_SHARED((N,), _f32),
            pltpu.VMEM_SHARED((N,), _f32),
            pltpu.SemaphoreType.DMA,
        ],
    )
    return kern(srcr, dstr, ewr, x3)


def _tc_enc_body(x_b, f1_b, f2_b, f3_b, w0, w1, w2, w3, b_ref, enc_ref):
    fb = pl.program_id(1)
    feats = (x_b, f1_b, f2_b, f3_b)
    ws = (w0, w1, w2, w3)
    part = sum(
        jnp.dot(f[0].astype(jnp.bfloat16), w[0].astype(jnp.bfloat16),
                preferred_element_type=_f32)
        for f, w in zip(feats, ws))

    @pl.when(fb == 0)
    def _():
        enc_ref[...] = part + b_ref[...]

    @pl.when(fb > 0)
    def _():
        enc_ref[...] += part


def _tc_enc(x3r, f1r, f2r, f3r, wb, b1):
    bm = 512
    fspec = lambda: pl.BlockSpec((1, bm, HID), lambda r, fb: (fb, r, 0))
    wspec = lambda k: pl.BlockSpec((1, HID, HID), lambda r, fb, k=k: (k * 16 + fb, 0, 0))
    return pl.pallas_call(
        _tc_enc_body,
        grid=(N // bm, 16),
        in_specs=[
            fspec(), fspec(), fspec(), fspec(),
            wspec(0), wspec(1), wspec(2), wspec(3),
            pl.BlockSpec((1, HID), lambda r, fb: (0, 0)),
        ],
        out_specs=pl.BlockSpec((bm, HID), lambda r, fb: (r, 0)),
        out_shape=jax.ShapeDtypeStruct((N, HID), _f32),
        compiler_params=_TC_PARAMS,
    )(x3r, f1r, f2r, f3r, wb, wb, wb, wb, b1)


def _sc_layer2(srcr, dstr, wr, zinit, h0):
    kern = pl.kernel(
        _sc_layer2_body,
        out_type=(
            jax.ShapeDtypeStruct((N, HID), _f32),
            jax.ShapeDtypeStruct((N, HID), _f32),
            jax.ShapeDtypeStruct((N, HID), _f32),
        ),
        mesh=plsc.VectorSubcoreMesh(**_SC_MESH),
        compiler_params=pltpu.CompilerParams(needs_layout_passes=False),
        scratch_types=[
            pltpu.VMEM((NCHUNK, CHUNK), _i32),
            pltpu.VMEM((NCHUNK, CHUNK), _i32),
            pltpu.VMEM((NCHUNK, CHUNK), _f32),
            pltpu.VMEM((CHUNK, HID), _f32),
            pltpu.VMEM_SHARED((N, HID), _f32),
            pltpu.SemaphoreType.DMA,
        ],
    )
    return kern(srcr, dstr, wr, zinit, h0)


_TC_PARAMS = pltpu.CompilerParams(vmem_limit_bytes=100 * 1024 * 1024)


def _tc1_body(x_ref, w_ref, b_ref, p_ref, h0_ref):
    p = jnp.dot(x_ref[...], w_ref[...],
                preferred_element_type=_f32, precision=lax.Precision.HIGHEST) + b_ref[...]
    p_ref[...] = p
    h0_ref[...] = p[:, K * HID:]


def _tc1(x, w1r, b1p):
    bm = 512
    return pl.pallas_call(
        _tc1_body,
        grid=(N // bm,),
        in_specs=[
            pl.BlockSpec((bm, IN_DIM), lambda r: (r, 0)),
            pl.BlockSpec((IN_DIM, (K + 1) * HID), lambda r: (0, 0)),
            pl.BlockSpec((1, (K + 1) * HID), lambda r: (0, 0)),
        ],
        out_specs=[
            pl.BlockSpec((bm, (K + 1) * HID), lambda r: (r, 0)),
            pl.BlockSpec((bm, HID), lambda r: (r, 0)),
        ],
        out_shape=[
            jax.ShapeDtypeStruct((N, (K + 1) * HID), _f32),
            jax.ShapeDtypeStruct((N, HID), _f32),
        ],
        compiler_params=_TC_PARAMS,
    )(x, w1r, b1p)


def _tc2_body(e_ref, g1_ref, g2_ref, g3_ref, w_ref, b_ref, zp_ref):
    cat = jnp.concatenate(
        [e_ref[...], g1_ref[...], g2_ref[...], g3_ref[...]], axis=1)
    z = jnp.dot(cat.astype(jnp.bfloat16), w_ref[...].astype(jnp.bfloat16),
                preferred_element_type=_f32) + b_ref[...]
    zp_ref[...] = jnp.concatenate(
        [z, jnp.zeros((z.shape[0], 1), _f32)], axis=1)


def _tc2(enc, g1, g2, g3, w2, b2):
    bm = 512
    blk = lambda: pl.BlockSpec((bm, HID), lambda r: (r, 0))
    return pl.pallas_call(
        _tc2_body,
        grid=(N // bm,),
        in_specs=[
            blk(), blk(), blk(), blk(),
            pl.BlockSpec(((K + 1) * HID, LAT), lambda r: (0, 0)),
            pl.BlockSpec((1, LAT), lambda r: (0, 0)),
        ],
        out_specs=pl.BlockSpec((bm, 16), lambda r: (r, 0)),
        out_shape=jax.ShapeDtypeStruct((N, 16), _f32),
        compiler_params=_TC_PARAMS,
    )(enc, g1, g2, g3, w2, b2)


def _tc3_body(zp_ref, wa_ref, ba_ref, ubar_ref, zbar_ref):
    z15 = zp_ref[:, :LAT].astype(jnp.bfloat16).astype(_f32)
    wa = wa_ref[...].astype(jnp.bfloat16).astype(_f32)
    ba = ba_ref[...]
    gmat = lax.dot_general(wa, wa, (((1,), (1,)), ((), ())),
                           preferred_element_type=_f32, precision=lax.Precision.HIGHEST)
    cvec = lax.dot_general(wa, ba, (((1,), (1,)), ((), ())),
                           preferred_element_type=_f32, precision=lax.Precision.HIGHEST)
    bb = jnp.sum(ba * ba)
    u = jnp.dot(z15, gmat, preferred_element_type=_f32, precision=lax.Precision.HIGHEST)
    s = jnp.dot(z15, cvec, preferred_element_type=_f32, precision=lax.Precision.HIGHEST)
    ones = jnp.ones((N, 1), _f32)
    zer = jnp.zeros((N, LAT), _f32)
    ubar_ref[...] = jnp.concatenate([u, s + bb, ones, zer], axis=1)
    zbar_ref[...] = jnp.concatenate([z15, ones, s, zer], axis=1)


def _tc3(zp, wa, ba):
    return pl.pallas_call(
        _tc3_body,
        out_shape=[
            jax.ShapeDtypeStruct((N, 32), _f32),
            jax.ShapeDtypeStruct((N, 32), _f32),
        ],
        compiler_params=_TC_PARAMS,
    )(zp, wa, ba)


def _tc4_body(u_ref, z_ref, o_ref):
    o_ref[...] = jax.nn.sigmoid(
        lax.dot_general(u_ref[...], z_ref[...], (((1,), (1,)), ((), ())),
                        preferred_element_type=_f32, precision=lax.Precision.HIGHEST))


def _tc4(ubar, zbar):
    bm, bn = 256, 1024
    return pl.pallas_call(
        _tc4_body,
        grid=(N // bm, N // bn),
        in_specs=[
            pl.BlockSpec((bm, 32), lambda i, j: (i, 0)),
            pl.BlockSpec((bn, 32), lambda i, j: (j, 0)),
        ],
        out_specs=pl.BlockSpec((bm, bn), lambda i, j: (i, j)),
        out_shape=jax.ShapeDtypeStruct((N, N), _f32),
        compiler_params=_TC_PARAMS,
    )(ubar, zbar)


def _tc5_body(zp_ref, mu_ref,
              wd1_ref, bd1_ref, wd2_ref, bd2_ref, wd3_ref, bd3_ref,
              wm_ref, bm_ref, wdsp_ref, bdsp_ref, wpi_ref, bpi_ref,
              mean_ref, disp_ref, pi_ref, q_ref):
    z15 = zp_ref[:, :LAT]
    h1 = jax.nn.relu(jnp.dot(z15, wd1_ref[...],
                             preferred_element_type=_f32, precision=lax.Precision.HIGHEST) + bd1_ref[...])
    h2 = jax.nn.relu(jnp.dot(h1, wd2_ref[...],
                             preferred_element_type=_f32, precision=lax.Precision.HIGHEST) + bd2_ref[...])
    h3 = jax.nn.relu(jnp.dot(h2, wd3_ref[...],
                             preferred_element_type=_f32, precision=lax.Precision.HIGHEST) + bd3_ref[...])
    mean_ref[...] = jnp.clip(
        jnp.exp(jnp.dot(h3, wm_ref[...], preferred_element_type=_f32, precision=lax.Precision.HIGHEST)
                + bm_ref[...]), 1e-5, 1e6)
    disp_ref[...] = jnp.clip(
        jax.nn.softplus(jnp.dot(h3, wdsp_ref[...],
                                preferred_element_type=_f32, precision=lax.Precision.HIGHEST) + bdsp_ref[...]),
        1e-4, 1e4)
    pi_ref[...] = jax.nn.sigmoid(
        jnp.dot(h3, wpi_ref[...], preferred_element_type=_f32, precision=lax.Precision.HIGHEST) + bpi_ref[...])
    mu = mu_ref[...]
    cross = lax.dot_general(z15, mu, (((1,), (1,)), ((), ())),
                            preferred_element_type=_f32, precision=lax.Precision.HIGHEST)
    z2 = jnp.sum(z15 * z15, axis=1, keepdims=True)
    m2 = jnp.sum(mu * mu, axis=1)[None, :]
    dist2 = z2 - 2.0 * cross + m2
    q = 1.0 / (1.0 + dist2)
    q_ref[...] = q / jnp.sum(q, axis=1, keepdims=True)


def _tc5(zp, mu, wd1, bd1, wd2, bd2, wd3, bd3, wm, bm_, wdsp, bdsp, wpi, bpi):
    bm = 512
    full = lambda shape: pl.BlockSpec(shape, lambda r: tuple(0 for _ in shape))
    return pl.pallas_call(
        _tc5_body,
        grid=(N // bm,),
        in_specs=[
            pl.BlockSpec((bm, 16), lambda r: (r, 0)),
            full((NCLUST, LAT)),
            full((LAT, 128)), full((1, 128)),
            full((128, 256)), full((1, 256)),
            full((256, 512)), full((1, 512)),
            full((512, IN_DIM)), full((1, IN_DIM)),
            full((512, IN_DIM)), full((1, IN_DIM)),
            full((512, IN_DIM)), full((1, IN_DIM)),
        ],
        out_specs=[
            pl.BlockSpec((bm, IN_DIM), lambda r: (r, 0)),
            pl.BlockSpec((bm, IN_DIM), lambda r: (r, 0)),
            pl.BlockSpec((bm, IN_DIM), lambda r: (r, 0)),
            pl.BlockSpec((bm, NCLUST), lambda r: (r, 0)),
        ],
        out_shape=[
            jax.ShapeDtypeStruct((N, IN_DIM), _f32),
            jax.ShapeDtypeStruct((N, IN_DIM), _f32),
            jax.ShapeDtypeStruct((N, IN_DIM), _f32),
            jax.ShapeDtypeStruct((N, NCLUST), _f32),
        ],
        compiler_params=_TC_PARAMS,
    )(zp, mu, wd1, bd1, wd2, bd2, wd3, bd3, wm, bm_, wdsp, bdsp, wpi, bpi)


def kernel(x_input, edge_index, edge_weight, W1, b1, W2, b2, W_adj, b_adj,
           mu, Wd1, bd1, Wd2, bd2, Wd3, bd3, Wm, bm, Wdsp, bdsp, Wpi, bpi):
    ei = edge_index.astype(_i32)
    srcr = ei[0].reshape(NTILES, NCHUNK, CHUNK)
    dstr = ei[1].reshape(NTILES, NCHUNK, CHUNK)
    ewr = edge_weight.reshape(NTILES, NCHUNK, CHUNK)

    x3 = jnp.pad(x_input, ((0, 0), (0, 16 * HID - IN_DIM)))
    x3 = x3.reshape(N, 16, HID).transpose(1, 0, 2)
    wb = jnp.pad(W1.reshape(K + 1, IN_DIM, HID),
                 ((0, 0), (0, 16 * HID - IN_DIM), (0, 0)))
    wb = wb.reshape((K + 1) * 16, HID, HID)
    zinit = jnp.zeros((N, HID), _f32)

    f1, f2, f3, w = _sc_wide(srcr, dstr, ewr, x3.reshape(16 * N, HID))
    enc = _tc_enc(x3, f1.reshape(16, N, HID), f2.reshape(16, N, HID),
                  f3.reshape(16, N, HID), wb, b1.reshape(1, -1))
    g1, g2, g3 = _sc_layer2(srcr, dstr, w, zinit, enc)
    zp = _tc2(enc, g1, g2, g3, W2, b2.reshape(1, -1))
    z = zp[:, :LAT]
    ubar, zbar = _tc3(zp, W_adj, b_adj.reshape(1, N))
    adj_out = _tc4(ubar, zbar)
    _mean, _disp, _pi, q = _tc5(zp, mu, Wd1, bd1.reshape(1, -1),
                                Wd2, bd2.reshape(1, -1), Wd3,
                                bd3.reshape(1, -1), Wm, bm.reshape(1, -1),
                                Wdsp, bdsp.reshape(1, -1), Wpi,
                                bpi.reshape(1, -1))
    return (adj_out, z, q, _mean, _disp, _pi)
